# Initial kernel scaffold; baseline (speedup 1.0000x reference)
import jax
import jax.numpy as jnp
from jax.experimental import pallas as pl


def kernel(x, edge_list, Wp1, bp1, Wl1, bl1, Wr1, Wp2, bp2, Wl2, bl2, Wr2, Wp3, bp3, Wl3, bl3, Wr3, Wp4, bp4, Wl4, bl4, Wr4, Wp5, bp5, Wl5, bl5, Wr5):
    raise NotImplementedError("write your pallas kernel here")


# speedup vs baseline: 3.6331x; 3.6331x over previous
import functools

import jax
import jax.numpy as jnp
from jax import lax
from jax.experimental import pallas as pl
from jax.experimental.pallas import tpu as pltpu
from jax.experimental.pallas import tpu_sc as plsc

_N = 10000
_E = 320000
_NPAD = 10240
_ROWS = _E // 128
_R = 1000
_f32 = jnp.float32


def _dot(a, b):
    return lax.dot_general(a, b, (((1,), (0,)), ((), ())),
                           preferred_element_type=_f32,
                           precision=lax.Precision.HIGHEST)


def _sc_aggregate(h0, h1, src2, dst2, H, with_cnt):
    mesh = plsc.VectorSubcoreMesh(core_axis_name="c", subcore_axis_name="s",
                                  num_cores=2, num_subcores=16)
    out_type = [jax.ShapeDtypeStruct((_N, H), _f32),
                jax.ShapeDtypeStruct((_N, H), _f32)]
    scratch = [pltpu.VMEM_SHARED((_NPAD, H), _f32),
               pltpu.VMEM((1, 128), jnp.int32),
               pltpu.VMEM((1, 128), jnp.int32),
               pltpu.VMEM((128, H), _f32)]
    if with_cnt:
        out_type.append(jax.ShapeDtypeStruct((_N,), _f32))
        scratch += [pltpu.VMEM_SHARED((_NPAD,), _f32),
                    pltpu.VMEM((128,), _f32),
                    pltpu.VMEM((640,), _f32)]

    def body(h0_ref, h1_ref, src_ref, dst_ref, agg0_ref, agg1_ref, *rest):
        if with_cnt:
            cnt_ref, acc, sidx, didx, rows, cnt_sp, ones_v, zc = rest
        else:
            acc, sidx, didx, rows = rest
        c = lax.axis_index("c")
        s = lax.axis_index("s")

        def _zrow(i, _):
            for j in range(H // 16):
                rows[i, pl.ds(j * 16, 16)] = jnp.zeros((16,), _f32)
            return 0
        lax.fori_loop(0, 128, _zrow, 0)
        for k in range(5):
            pltpu.sync_copy(rows, acc.at[pl.ds(s * 640 + k * 128, 128)])
        if with_cnt:
            def _zc(i, _):
                zc[pl.ds(i * 16, 16)] = jnp.zeros((16,), _f32)
                ones_v[pl.ds((i % 8) * 16, 16)] = jnp.ones((16,), _f32)
                return 0
            lax.fori_loop(0, 40, _zc, 0)
            pltpu.sync_copy(zc, cnt_sp.at[pl.ds(s * 640, 640)])
        plsc.subcore_barrier()

        base = s * 156 + jnp.minimum(s, 4)
        nrows = 156 + (s < 4).astype(jnp.int32)

        def edge_pass(h_ref, do_cnt):
            def step(i, _):
                r = base + i
                pltpu.sync_copy(src_ref.at[pl.ds(r, 1)], sidx)
                pltpu.sync_copy(dst_ref.at[pl.ds(r, 1)], didx)
                pltpu.sync_copy(h_ref.at[sidx.at[0]], rows)
                pltpu.sync_copy(rows, acc.at[didx.at[0]], add=True)
                if do_cnt:
                    pltpu.sync_copy(ones_v, cnt_sp.at[didx.at[0]], add=True)
                return 0
            lax.fori_loop(0, nrows, step, 0)

        @pl.when(c == 0)
        def _():
            edge_pass(h0_ref, with_cnt)

        @pl.when(c == 1)
        def _():
            edge_pass(h1_ref, False)

        plsc.subcore_barrier()

        def copy_out(agg_ref):
            @pl.when(s < 15)
            def _():
                pltpu.sync_copy(acc.at[pl.ds(s * 640, 640)],
                                agg_ref.at[pl.ds(s * 640, 640)])

            @pl.when(s == 15)
            def _():
                pltpu.sync_copy(acc.at[pl.ds(9600, 400)],
                                agg_ref.at[pl.ds(9600, 400)])

        @pl.when(c == 0)
        def _():
            copy_out(agg0_ref)
            if with_cnt:
                @pl.when(s == 0)
                def _():
                    pltpu.sync_copy(cnt_sp.at[pl.ds(0, _N)], cnt_ref)

        @pl.when(c == 1)
        def _():
            copy_out(agg1_ref)

    kfn = pl.kernel(body, out_type=out_type, mesh=mesh, scratch_types=scratch,
                    compiler_params=pltpu.CompilerParams(
                        use_tc_tiling_on_sc=False))
    return kfn(h0, h1, src2, dst2)


def _project_tc(x, WpT, bp2, H):
    din = x.shape[1]

    def body(x_ref, w_ref, b_ref, h0_ref, h1_ref):
        h = jnp.maximum(_dot(x_ref[...], w_ref[...]) + b_ref[...], 0.0)
        h0_ref[...] = h[:, :H]
        h1_ref[...] = h[:, H:]

    return pl.pallas_call(
        body,
        grid=(_N // _R,),
        in_specs=[pl.BlockSpec((_R, din), lambda i: (i, 0)),
                  pl.BlockSpec((din, din), lambda i: (0, 0)),
                  pl.BlockSpec((1, din), lambda i: (0, 0))],
        out_specs=[pl.BlockSpec((_R, H), lambda i: (i, 0)),
                   pl.BlockSpec((_R, H), lambda i: (i, 0))],
        out_shape=[jax.ShapeDtypeStruct((_N, H), _f32),
                   jax.ShapeDtypeStruct((_N, H), _f32)],
    )(x, WpT, bp2)


def _combine_tc(agg0, agg1, cnt2, x, Wl0T, Wl1T, bl2, WrT, WpTn, bpn2):
    H = agg0.shape[1]
    din = x.shape[1]
    dout = WrT.shape[1]
    project = WpTn is not None
    Hn = dout // 2

    def body(a0, a1, cnt, xr, wl0, wl1, bl, wr, *rest):
        if project:
            wpn, bpn, out_ref, h0_ref, h1_ref = rest
        else:
            out_ref, = rest
        scale = 1.0 / jnp.maximum(cnt[...], 1.0)
        out = (_dot(a0[...] * scale, wl0[...]) +
               _dot(a1[...] * scale, wl1[...]) +
               _dot(xr[...], wr[...]) + bl[...])
        nrm = jnp.sqrt(jnp.sum(out * out, axis=1, keepdims=True))
        y = out / jnp.maximum(nrm, 1e-12)
        out_ref[...] = y
        if project:
            hn = jnp.maximum(_dot(y, wpn[...]) + bpn[...], 0.0)
            h0_ref[...] = hn[:, :Hn]
            h1_ref[...] = hn[:, Hn:]

    in_specs = [pl.BlockSpec((_R, H), lambda i: (i, 0)),
                pl.BlockSpec((_R, H), lambda i: (i, 0)),
                pl.BlockSpec((_R, 1), lambda i: (i, 0)),
                pl.BlockSpec((_R, din), lambda i: (i, 0)),
                pl.BlockSpec((H, dout), lambda i: (0, 0)),
                pl.BlockSpec((H, dout), lambda i: (0, 0)),
                pl.BlockSpec((1, dout), lambda i: (0, 0)),
                pl.BlockSpec((din, dout), lambda i: (0, 0))]
    out_specs = [pl.BlockSpec((_R, dout), lambda i: (i, 0))]
    out_shape = [jax.ShapeDtypeStruct((_N, dout), _f32)]
    args = [agg0, agg1, cnt2, x, Wl0T, Wl1T, bl2, WrT]
    if project:
        in_specs += [pl.BlockSpec((dout, dout), lambda i: (0, 0)),
                     pl.BlockSpec((1, dout), lambda i: (0, 0))]
        out_specs += [pl.BlockSpec((_R, Hn), lambda i: (i, 0)),
                      pl.BlockSpec((_R, Hn), lambda i: (i, 0))]
        out_shape += [jax.ShapeDtypeStruct((_N, Hn), _f32),
                      jax.ShapeDtypeStruct((_N, Hn), _f32)]
        args += [WpTn, bpn2]

    return pl.pallas_call(
        body,
        grid=(_N // _R,),
        in_specs=in_specs,
        out_specs=out_specs,
        out_shape=out_shape,
    )(*args)


def kernel(x, edge_list,
           Wp1, bp1, Wl1, bl1, Wr1,
           Wp2, bp2, Wl2, bl2, Wr2,
           Wp3, bp3, Wl3, bl3, Wr3,
           Wp4, bp4, Wl4, bl4, Wr4,
           Wp5, bp5, Wl5, bl5, Wr5):
    layers = [(Wp1, bp1, Wl1, bl1, Wr1), (Wp2, bp2, Wl2, bl2, Wr2),
              (Wp3, bp3, Wl3, bl3, Wr3), (Wp4, bp4, Wl4, bl4, Wr4),
              (Wp5, bp5, Wl5, bl5, Wr5)]
    ei = edge_list.T
    src2 = ei[0].reshape(_ROWS, 128)
    dst2 = ei[1].reshape(_ROWS, 128)

    Wp, bp = layers[0][0], layers[0][1]
    h0, h1 = _project_tc(x, Wp.T, bp.reshape(1, -1), Wp.shape[0] // 2)

    h_cur = x
    cnt2 = None
    for i in range(5):
        Wp, bp, Wl, bl, Wr = layers[i]
        H = Wp.shape[0] // 2
        if i == 0:
            agg0, agg1, cnt = _sc_aggregate(h0, h1, src2, dst2, H, True)
            cnt2 = cnt.reshape(_N, 1)
        else:
            agg0, agg1 = _sc_aggregate(h0, h1, src2, dst2, H, False)
        WlT = Wl.T
        if i < 4:
            Wpn, bpn = layers[i + 1][0], layers[i + 1][1]
            h_cur, h0, h1 = _combine_tc(agg0, agg1, cnt2, h_cur,
                                        WlT[:H], WlT[H:], bl.reshape(1, -1),
                                        Wr.T, Wpn.T, bpn.reshape(1, -1))
        else:
            h_cur, = _combine_tc(agg0, agg1, cnt2, h_cur,
                                 WlT[:H], WlT[H:], bl.reshape(1, -1),
                                 Wr.T, None, None)
    return h_cur
